# baseline (speedup 1.0000x reference)
import functools

import jax
import jax.numpy as jnp
from jax import lax
from jax.experimental import pallas as pl
from jax.experimental.pallas import tpu as pltpu

_F32 = jnp.float32
_BF16 = jnp.bfloat16


def _split(a):
    hi = a.astype(_BF16)
    return hi, (a - hi.astype(_F32)).astype(_BF16)


def _mm3(a, b, dims=(((1,), (0,)), ((), ()))):
    ah, al = _split(a)
    bh, bl = _split(b)

    def d(u, v):
        return lax.dot_general(u, v, dims, preferred_element_type=_F32)

    return d(ah, bh) + d(ah, bl) + d(al, bh)


def _mm1(a, b, dims=(((1,), (0,)), ((), ()))):
    return lax.dot_general(a.astype(_BF16), b.astype(_BF16), dims,
                           preferred_element_type=_F32)


_DIMS_T = (((1,), (1,)), ((), ()))


def _pool_axis(y, axis):
    w = y.shape[axis] - 2

    def sl(s, e):
        return lax.slice_in_dim(y, s, e, axis=axis)

    m = jnp.maximum(jnp.maximum(sl(0, w), sl(1, w + 1)), sl(2, w + 2))
    if w % 2 == 1:
        m = jnp.concatenate([m, lax.slice_in_dim(m, 0, 1, axis=axis)],
                            axis=axis)
    shape = list(m.shape)
    shape[axis:axis + 1] = [shape[axis] // 2, 2]
    m = m.reshape(shape)
    return lax.index_in_dim(m, 0, axis=axis + 1, keepdims=False)

def _conv_body(x_ref, w_ref, b_ref, o_ref, acc_ref, *, kh, kw, ho, wo, nb, co,
               pool):
    c = x_ref.shape[-1]
    acc_ref[...] = jnp.zeros_like(acc_ref)

    def tap(t, carry):
        a, b = t // kw, t % kw
        xs = x_ref[pl.ds(a, ho), pl.ds(b, wo), :, :]
        xs = xs.reshape(ho * wo * nb, c)
        acc_ref[...] += _mm1(xs, w_ref[t, :, :])
        return carry

    lax.fori_loop(0, kh * kw, tap, 0)
    y = jnp.maximum(acc_ref[...] + b_ref[...], 0.0)
    y = y.reshape(ho, wo, nb, co)
    if pool:
        y = _pool_axis(_pool_axis(y, 0), 1)
    o_ref[...] = y


def _conv(x, w, b, kh, kw, nb, pool):
    hp, wp, n, c = x.shape
    t, _, co = w.shape
    ho, wo = hp - kh + 1, wp - kw + 1
    if pool:
        oh = ow = (ho - 3) // 2 + 1
    else:
        oh, ow = ho, wo
    body = functools.partial(_conv_body, kh=kh, kw=kw, ho=ho, wo=wo, nb=nb,
                             co=co, pool=pool)
    return pl.pallas_call(
        body,
        grid=(n // nb,),
        in_specs=[
            pl.BlockSpec((hp, wp, nb, c), lambda i: (0, 0, i, 0)),
            pl.BlockSpec((t, c, co), lambda i: (0, 0, 0)),
            pl.BlockSpec((1, co), lambda i: (0, 0)),
        ],
        out_specs=pl.BlockSpec((oh, ow, nb, co), lambda i: (0, 0, i, 0)),
        out_shape=jax.ShapeDtypeStruct((oh, ow, n, co), _F32),
        scratch_shapes=[pltpu.VMEM((ho * wo * nb, co), _F32)],
    )(x, w, b.reshape(1, co))


def _conv1_body(xa_ref, xb_ref, w_ref, b_ref, o_ref, acc_ref, *, hb, wo, nb,
                co):
    c = xa_ref.shape[-1]
    acc_ref[...] = jnp.zeros_like(acc_ref)
    for a in range(3):
        for b in range(3):
            if a == 0:
                xs = xa_ref[:, b:b + wo, :, :]
            else:
                xs = jnp.concatenate(
                    [xa_ref[a:hb, b:b + wo, :, :], xb_ref[0:a, b:b + wo, :, :]],
                    axis=0)
            xs = xs.reshape(hb * wo * nb, c)
            acc_ref[...] += _mm1(xs, w_ref[3 * a + b, :, :])
    y = jnp.maximum(acc_ref[...] + b_ref[...], 0.0)
    o_ref[...] = y.reshape(hb, wo, nb, co)


def _conv1(x, w, b, nb, hb=8):
    hp, wp, n, c = x.shape
    co = w.shape[-1]
    wo = wp - 2
    nh = hp // hb - 1
    body = functools.partial(_conv1_body, hb=hb, wo=wo, nb=nb, co=co)
    return pl.pallas_call(
        body,
        grid=(n // nb, nh),
        in_specs=[
            pl.BlockSpec((hb, wp, nb, c), lambda i, j: (j, 0, i, 0)),
            pl.BlockSpec((hb, wp, nb, c), lambda i, j: (j + 1, 0, i, 0)),
            pl.BlockSpec((9, c, co), lambda i, j: (0, 0, 0)),
            pl.BlockSpec((1, co), lambda i, j: (0, 0)),
        ],
        out_specs=pl.BlockSpec((hb, wo, nb, co), lambda i, j: (j, 0, i, 0)),
        out_shape=jax.ShapeDtypeStruct((nh * hb, wo, n, co), _F32),
        scratch_shapes=[pltpu.VMEM((hb * wo * nb, co), _F32)],
    )(x, x, w, b.reshape(1, co))


def _pool_body(x_ref, o_ref):
    o_ref[...] = _pool_axis(_pool_axis(x_ref[...], 0), 1)


def _pool(x, nb):
    h, w, n, c = x.shape
    ph, pw = (h - 3) // 2 + 1, (w - 3) // 2 + 1
    return pl.pallas_call(
        _pool_body,
        grid=(n // nb,),
        in_specs=[pl.BlockSpec((h, w, nb, c), lambda i: (0, 0, i, 0))],
        out_specs=pl.BlockSpec((ph, pw, nb, c), lambda i: (0, 0, i, 0)),
        out_shape=jax.ShapeDtypeStruct((ph, pw, n, c), _F32),
    )(x)


def _fc(x, w, b, relu, bo=None, bk=None):
    m, kdim = x.shape
    o = w.shape[0]
    bo = bo or o
    bk = bk or kdim
    no, nk = o // bo, kdim // bk

    def body(x_ref, w_ref, b_ref, o_ref, acc_ref):
        kk = pl.program_id(1)
        part = _mm1(x_ref[...], w_ref[...], _DIMS_T)

        @pl.when(kk == 0)
        def _():
            acc_ref[...] = part

        @pl.when(kk > 0)
        def _():
            acc_ref[...] += part

        @pl.when(kk == nk - 1)
        def _():
            y = acc_ref[...] + b_ref[...]
            if relu:
                y = jnp.maximum(y, 0.0)
            o_ref[...] = y

    return pl.pallas_call(
        body,
        grid=(no, nk),
        in_specs=[
            pl.BlockSpec((m, bk), lambda i, j: (0, j)),
            pl.BlockSpec((bo, bk), lambda i, j: (i, j)),
            pl.BlockSpec((1, bo), lambda i, j: (0, i)),
        ],
        out_specs=pl.BlockSpec((m, bo), lambda i, j: (0, i)),
        out_shape=jax.ShapeDtypeStruct((m, o), _F32),
        scratch_shapes=[pltpu.VMEM((m, bo), _F32)],
    )(x, w, b.reshape(1, o))


def _vq(latent, emb):
    n, dm = latent.shape
    ne = emb.shape[0]

    def body(x_ref, e_ref, loss_ref, q_ref, perp_ref):
        x = x_ref[...]
        e = e_ref[...]
        x2 = jnp.sum(x * x, axis=1, keepdims=True)
        e2 = lax.dot_general(jnp.ones((1, dm), _F32), e * e, _DIMS_T,
                             precision=lax.Precision.HIGHEST,
                             preferred_element_type=_F32)
        xe = _mm1(x, e, _DIMS_T)
        d = x2 + e2 - 2.0 * xe
        iota = lax.broadcasted_iota(jnp.int32, (n, ne), 1)
        dmin = jnp.min(d, axis=1, keepdims=True)
        idx = jnp.min(jnp.where(d == dmin, iota, ne), axis=1, keepdims=True)
        enc = (iota == idx).astype(_F32)
        q = _mm1(enc, e)
        diff = q - x
        ss = jnp.sum(jnp.sum(diff * diff, axis=1, keepdims=True), axis=0,
                     keepdims=True)
        loss_ref[...] = 0.25 * ss / (n * dm)
        q_ref[...] = q
        avg = jnp.sum(enc, axis=0, keepdims=True) / n
        ent = jnp.sum(avg * jnp.log(avg + 1e-10), axis=1, keepdims=True)
        perp_ref[...] = jnp.exp(-ent)

    loss, q, perp = pl.pallas_call(
        body,
        out_shape=(jax.ShapeDtypeStruct((1, 1), _F32),
                   jax.ShapeDtypeStruct((n, dm), _F32),
                   jax.ShapeDtypeStruct((1, 1), _F32)),
    )(latent, emb)
    return loss.reshape(()), q, perp.reshape(())


def kernel(x, pose, img, img_crop, img_zoom, params):
    p = params
    w1 = jnp.pad(p["c1w"], ((0, 0), (0, 0), (0, 1), (0, 1)))
    w1 = w1.reshape(64, 3, 3, 4, 3, 4).transpose(2, 4, 3, 5, 1, 0)
    w1 = w1.reshape(9, 48, 64)
    w2 = p["c2w"].transpose(2, 3, 1, 0).reshape(25, 64, 192)
    w3 = p["c3w"].transpose(2, 3, 1, 0).reshape(9, 192, 384)
    w4 = p["c4w"].transpose(2, 3, 1, 0).reshape(9, 384, 256)
    w5 = p["c5w"].transpose(2, 3, 1, 0).reshape(9, 256, 256)
    feats = []
    for im in (img, img_crop, img_zoom):
        im = jnp.pad(im, ((0, 0), (0, 0), (2, 30), (2, 2)))
        xd = im.reshape(16, 3, 64, 4, 57, 4).transpose(2, 4, 0, 3, 5, 1)
        xd = xd.reshape(64, 57, 16, 48)
        y = _conv1(xd, w1, p["c1b"], nb=8)
        y = _pool(y, nb=8)
        y = jnp.pad(y, ((2, 2), (2, 2), (0, 0), (0, 0)))
        y = _conv(y, w2, p["c2b"], 5, 5, nb=8, pool=True)
        y = jnp.pad(y, ((1, 1), (1, 1), (0, 0), (0, 0)))
        y = _conv(y, w3, p["c3b"], 3, 3, nb=16, pool=False)
        y = jnp.pad(y, ((1, 1), (1, 1), (0, 0), (0, 0)))
        y = _conv(y, w4, p["c4b"], 3, 3, nb=16, pool=False)
        y = jnp.pad(y, ((1, 1), (1, 1), (0, 0), (0, 0)))
        y = _conv(y, w5, p["c5b"], 3, 3, nb=16, pool=True)
        feats.append(y.transpose(2, 3, 0, 1).reshape(16, 9216))
    feat = jnp.concatenate(feats, axis=0)
    f = _fc(feat, p["fc6w"], p["fc6b"], True, bo=512, bk=2304)
    f = _fc(f, p["fc7w"], p["fc7b"], True, bo=512, bk=2048)
    f1, f2, f3 = f[0:16], f[16:32], f[32:48]
    pf = _fc(pose, p["ce_fc1w"], p["ce_fc1b"], True)
    hcat = jnp.concatenate([pf, f1, f2, f3], axis=1)
    c = _fc(hcat, p["ce_fc2w"], p["ce_fc2b"], True, bo=512, bk=3328)
    h = _fc(x, p["e_fc1w"], p["e_fc1b"], True)
    h = _fc(h, p["e_fc2w"], p["e_fc2b"], True)
    latent = _fc(jnp.concatenate([h, c], axis=1), p["e_flw"], p["e_flb"], False)
    loss, q, perp = _vq(latent, p["emb"])
    d1 = _fc(q, p["d_fc1w"], p["d_fc1b"], True)
    d2 = _fc(d1, p["d_fc2w"], p["d_fc2b"], True)
    c2 = _fc(c, p["d_fc3w"], p["d_fc3b"], True)
    d4 = _fc(jnp.concatenate([d2, c2], axis=1), p["d_fc4w"], p["d_fc4b"], True)
    d5 = _fc(d4, p["d_fc5w"], p["d_fc5b"], True)
    xr = _fc(d5, p["d_fc6w"], p["d_fc6b"], False)
    return loss, xr, perp


# speedup vs baseline: 1.0916x; 1.0916x over previous
import functools

import jax
import jax.numpy as jnp
from jax import lax
from jax.experimental import pallas as pl
from jax.experimental.pallas import tpu as pltpu

_F32 = jnp.float32
_BF16 = jnp.bfloat16


def _split(a):
    hi = a.astype(_BF16)
    return hi, (a - hi.astype(_F32)).astype(_BF16)


def _mm3(a, b, dims=(((1,), (0,)), ((), ()))):
    ah, al = _split(a)
    bh, bl = _split(b)

    def d(u, v):
        return lax.dot_general(u, v, dims, preferred_element_type=_F32)

    return d(ah, bh) + d(ah, bl) + d(al, bh)


def _mm1(a, b, dims=(((1,), (0,)), ((), ()))):
    return lax.dot_general(a.astype(_BF16), b.astype(_BF16), dims,
                           preferred_element_type=_F32)


_DIMS_T = (((1,), (1,)), ((), ()))


def _pool_axis(y, axis):
    w = y.shape[axis] - 2

    def sl(s, e):
        return lax.slice_in_dim(y, s, e, axis=axis)

    m = jnp.maximum(jnp.maximum(sl(0, w), sl(1, w + 1)), sl(2, w + 2))
    if w % 2 == 1:
        m = jnp.concatenate([m, lax.slice_in_dim(m, 0, 1, axis=axis)],
                            axis=axis)
    shape = list(m.shape)
    shape[axis:axis + 1] = [shape[axis] // 2, 2]
    m = m.reshape(shape)
    return lax.index_in_dim(m, 0, axis=axis + 1, keepdims=False)

def _conv_body(x_ref, w_ref, b_ref, o_ref, acc_ref, *, kh, kw, ho, wo, nb, co,
               pool, pack):
    c = x_ref.shape[-1]
    acc_ref[...] = jnp.zeros_like(acc_ref)

    if pack:
        prs = (kw + 1) // 2

        def tap(q, carry):
            a, pr = q // prs, q % prs
            x0 = x_ref[pl.ds(a, ho), pl.ds(2 * pr, wo), :, :]
            x1 = x_ref[pl.ds(a, ho), pl.ds(2 * pr + 1, wo), :, :]
            xs = jnp.concatenate([x0, x1], axis=-1)
            xs = xs.reshape(ho * wo * nb, 2 * c)
            acc_ref[...] += _mm1(xs, w_ref[q, :, :])
            return carry

        lax.fori_loop(0, kh * prs, tap, 0)
    else:
        def tap(t, carry):
            a, b = t // kw, t % kw
            xs = x_ref[pl.ds(a, ho), pl.ds(b, wo), :, :]
            xs = xs.reshape(ho * wo * nb, c)
            acc_ref[...] += _mm1(xs, w_ref[t, :, :])
            return carry

        lax.fori_loop(0, kh * kw, tap, 0)
    y = jnp.maximum(acc_ref[...] + b_ref[...], 0.0)
    y = y.reshape(ho, wo, nb, co)
    if pool:
        y = _pool_axis(_pool_axis(y, 0), 1)
    o_ref[...] = y


def _conv(x, w, b, kh, kw, nb, pool, pack=False):
    hp, wp, n, c = x.shape
    t, ck, co = w.shape
    ho, wo = hp - kh + 1, wp - kw + 1
    if pack:
        wo = wp - kw
    if pool:
        oh = ow = (ho - 3) // 2 + 1
    else:
        oh, ow = ho, wo
    body = functools.partial(_conv_body, kh=kh, kw=kw, ho=ho, wo=wo, nb=nb,
                             co=co, pool=pool, pack=pack)
    return pl.pallas_call(
        body,
        grid=(n // nb,),
        in_specs=[
            pl.BlockSpec((hp, wp, nb, c), lambda i: (0, 0, i, 0)),
            pl.BlockSpec((t, ck, co), lambda i: (0, 0, 0)),
            pl.BlockSpec((1, co), lambda i: (0, 0)),
        ],
        out_specs=pl.BlockSpec((oh, ow, nb, co), lambda i: (0, 0, i, 0)),
        out_shape=jax.ShapeDtypeStruct((oh, ow, n, co), _F32),
        scratch_shapes=[pltpu.VMEM((ho * wo * nb, co), _F32)],
    )(x, w, b.reshape(1, co))


def _conv1_body(xa_ref, xb_ref, w_ref, b_ref, o_ref, acc_ref, *, hb, wo, nb,
                co):
    c = xa_ref.shape[-1]
    acc_ref[...] = jnp.zeros_like(acc_ref)

    def sl(t):
        a, b = t // 3, t % 3
        if a == 0:
            return xa_ref[:, b:b + wo, :, :]
        return jnp.concatenate(
            [xa_ref[a:hb, b:b + wo, :, :], xb_ref[0:a, b:b + wo, :, :]],
            axis=0)

    for q in range(5):
        x0 = sl(2 * q)
        x1 = sl(2 * q + 1) if 2 * q + 1 < 9 else x0
        xs = jnp.concatenate([x0, x1], axis=-1).reshape(hb * wo * nb, 2 * c)
        acc_ref[...] += _mm1(xs, w_ref[q, :, :])
    y = jnp.maximum(acc_ref[...] + b_ref[...], 0.0)
    o_ref[...] = y.reshape(hb, wo, nb, co)


def _conv1(x, w, b, nb, hb=8):
    hp, wp, n, c = x.shape
    t, ck, co = w.shape
    wo = wp - 2
    nh = hp // hb - 1
    body = functools.partial(_conv1_body, hb=hb, wo=wo, nb=nb, co=co)
    return pl.pallas_call(
        body,
        grid=(n // nb, nh),
        in_specs=[
            pl.BlockSpec((hb, wp, nb, c), lambda i, j: (j, 0, i, 0)),
            pl.BlockSpec((hb, wp, nb, c), lambda i, j: (j + 1, 0, i, 0)),
            pl.BlockSpec((t, ck, co), lambda i, j: (0, 0, 0)),
            pl.BlockSpec((1, co), lambda i, j: (0, 0)),
        ],
        out_specs=pl.BlockSpec((hb, wo, nb, co), lambda i, j: (j, 0, i, 0)),
        out_shape=jax.ShapeDtypeStruct((nh * hb, wo, n, co), _F32),
        scratch_shapes=[pltpu.VMEM((hb * wo * nb, co), _F32)],
    )(x, x, w, b.reshape(1, co))


def _pool_body(x_ref, o_ref):
    o_ref[...] = _pool_axis(_pool_axis(x_ref[...], 0), 1)


def _pool(x, nb):
    h, w, n, c = x.shape
    ph, pw = (h - 3) // 2 + 1, (w - 3) // 2 + 1
    return pl.pallas_call(
        _pool_body,
        grid=(n // nb,),
        in_specs=[pl.BlockSpec((h, w, nb, c), lambda i: (0, 0, i, 0))],
        out_specs=pl.BlockSpec((ph, pw, nb, c), lambda i: (0, 0, i, 0)),
        out_shape=jax.ShapeDtypeStruct((ph, pw, n, c), _F32),
    )(x)


def _fc(x, w, b, relu, bo=None, bk=None):
    m, kdim = x.shape
    o = w.shape[0]
    bo = bo or o
    bk = bk or kdim
    no, nk = o // bo, kdim // bk

    def body(x_ref, w_ref, b_ref, o_ref, acc_ref):
        kk = pl.program_id(1)
        part = _mm1(x_ref[...], w_ref[...], _DIMS_T)

        @pl.when(kk == 0)
        def _():
            acc_ref[...] = part

        @pl.when(kk > 0)
        def _():
            acc_ref[...] += part

        @pl.when(kk == nk - 1)
        def _():
            y = acc_ref[...] + b_ref[...]
            if relu:
                y = jnp.maximum(y, 0.0)
            o_ref[...] = y

    return pl.pallas_call(
        body,
        grid=(no, nk),
        in_specs=[
            pl.BlockSpec((m, bk), lambda i, j: (0, j)),
            pl.BlockSpec((bo, bk), lambda i, j: (i, j)),
            pl.BlockSpec((1, bo), lambda i, j: (0, i)),
        ],
        out_specs=pl.BlockSpec((m, bo), lambda i, j: (0, i)),
        out_shape=jax.ShapeDtypeStruct((m, o), _F32),
        scratch_shapes=[pltpu.VMEM((m, bo), _F32)],
    )(x, w, b.reshape(1, o))


def _vq(latent, emb):
    n, dm = latent.shape
    ne = emb.shape[0]

    def body(x_ref, e_ref, loss_ref, q_ref, perp_ref):
        x = x_ref[...]
        e = e_ref[...]
        x2 = jnp.sum(x * x, axis=1, keepdims=True)
        e2 = lax.dot_general(jnp.ones((1, dm), _F32), e * e, _DIMS_T,
                             precision=lax.Precision.HIGHEST,
                             preferred_element_type=_F32)
        xe = _mm1(x, e, _DIMS_T)
        d = x2 + e2 - 2.0 * xe
        iota = lax.broadcasted_iota(jnp.int32, (n, ne), 1)
        dmin = jnp.min(d, axis=1, keepdims=True)
        idx = jnp.min(jnp.where(d == dmin, iota, ne), axis=1, keepdims=True)
        enc = (iota == idx).astype(_F32)
        q = _mm1(enc, e)
        diff = q - x
        ss = jnp.sum(jnp.sum(diff * diff, axis=1, keepdims=True), axis=0,
                     keepdims=True)
        loss_ref[...] = 0.25 * ss / (n * dm)
        q_ref[...] = q
        avg = jnp.sum(enc, axis=0, keepdims=True) / n
        ent = jnp.sum(avg * jnp.log(avg + 1e-10), axis=1, keepdims=True)
        perp_ref[...] = jnp.exp(-ent)

    loss, q, perp = pl.pallas_call(
        body,
        out_shape=(jax.ShapeDtypeStruct((1, 1), _F32),
                   jax.ShapeDtypeStruct((n, dm), _F32),
                   jax.ShapeDtypeStruct((1, 1), _F32)),
    )(latent, emb)
    return loss.reshape(()), q, perp.reshape(())


def kernel(x, pose, img, img_crop, img_zoom, params):
    p = params
    imgs = jnp.concatenate([img, img_crop, img_zoom], axis=0)
    imgs = jnp.pad(imgs, ((0, 0), (0, 1), (2, 30), (2, 2)))
    xd = imgs.reshape(48, 4, 64, 4, 57, 4).transpose(2, 4, 0, 3, 5, 1)
    xd = xd.reshape(64, 57, 48, 64)
    w1 = jnp.pad(p["c1w"], ((0, 0), (0, 1), (0, 1), (0, 1)))
    w1 = w1.reshape(64, 4, 3, 4, 3, 4).transpose(2, 4, 3, 5, 1, 0)
    w1 = jnp.pad(w1.reshape(9, 64, 64), ((0, 1), (0, 0), (0, 0)))
    w1 = w1.reshape(5, 128, 64)
    y = _conv1(xd, w1, p["c1b"], nb=8)
    y = _pool(y, nb=8)
    y = jnp.pad(y, ((2, 2), (2, 3), (0, 0), (0, 0)))
    w2 = p["c2w"].transpose(2, 3, 1, 0)
    w2 = jnp.pad(w2, ((0, 0), (0, 1), (0, 0), (0, 0)))
    w2 = w2.reshape(15, 128, 192)
    y = _conv(y, w2, p["c2b"], 5, 5, nb=8, pool=True, pack=True)
    y = jnp.pad(y, ((1, 1), (1, 1), (0, 0), (0, 0)))
    w3 = p["c3w"].transpose(2, 3, 1, 0).reshape(9, 192, 384)
    y = _conv(y, w3, p["c3b"], 3, 3, nb=16, pool=False)
    y = jnp.pad(y, ((1, 1), (1, 1), (0, 0), (0, 0)))
    w4 = p["c4w"].transpose(2, 3, 1, 0).reshape(9, 384, 256)
    y = _conv(y, w4, p["c4b"], 3, 3, nb=16, pool=False)
    y = jnp.pad(y, ((1, 1), (1, 1), (0, 0), (0, 0)))
    w5 = p["c5w"].transpose(2, 3, 1, 0).reshape(9, 256, 256)
    y = _conv(y, w5, p["c5b"], 3, 3, nb=16, pool=True)
    feat = y.transpose(2, 3, 0, 1).reshape(48, 9216)
    f = _fc(feat, p["fc6w"], p["fc6b"], True, bo=512, bk=2304)
    f = _fc(f, p["fc7w"], p["fc7b"], True, bo=512, bk=2048)
    f1, f2, f3 = f[0:16], f[16:32], f[32:48]
    pf = _fc(pose, p["ce_fc1w"], p["ce_fc1b"], True)
    hcat = jnp.concatenate([pf, f1, f2, f3], axis=1)
    c = _fc(hcat, p["ce_fc2w"], p["ce_fc2b"], True, bo=512, bk=3328)
    h = _fc(x, p["e_fc1w"], p["e_fc1b"], True)
    h = _fc(h, p["e_fc2w"], p["e_fc2b"], True)
    latent = _fc(jnp.concatenate([h, c], axis=1), p["e_flw"], p["e_flb"], False)
    loss, q, perp = _vq(latent, p["emb"])
    d1 = _fc(q, p["d_fc1w"], p["d_fc1b"], True)
    d2 = _fc(d1, p["d_fc2w"], p["d_fc2b"], True)
    c2 = _fc(c, p["d_fc3w"], p["d_fc3b"], True)
    d4 = _fc(jnp.concatenate([d2, c2], axis=1), p["d_fc4w"], p["d_fc4b"], True)
    d5 = _fc(d4, p["d_fc5w"], p["d_fc5b"], True)
    xr = _fc(d5, p["d_fc6w"], p["d_fc6b"], False)
    return loss, xr, perp


# speedup vs baseline: 1.1099x; 1.0168x over previous
import functools

import jax
import jax.numpy as jnp
from jax import lax
from jax.experimental import pallas as pl
from jax.experimental.pallas import tpu as pltpu

_F32 = jnp.float32
_BF16 = jnp.bfloat16


def _split(a):
    hi = a.astype(_BF16)
    return hi, (a - hi.astype(_F32)).astype(_BF16)


def _mm3(a, b, dims=(((1,), (0,)), ((), ()))):
    ah, al = _split(a)
    bh, bl = _split(b)

    def d(u, v):
        return lax.dot_general(u, v, dims, preferred_element_type=_F32)

    return d(ah, bh) + d(ah, bl) + d(al, bh)


def _mm1(a, b, dims=(((1,), (0,)), ((), ()))):
    return lax.dot_general(a.astype(_BF16), b.astype(_BF16), dims,
                           preferred_element_type=_F32)


_DIMS_T = (((1,), (1,)), ((), ()))


def _pool_axis(y, axis):
    w = y.shape[axis] - 2

    def sl(s, e):
        return lax.slice_in_dim(y, s, e, axis=axis)

    m = jnp.maximum(jnp.maximum(sl(0, w), sl(1, w + 1)), sl(2, w + 2))
    if w % 2 == 1:
        m = jnp.concatenate([m, lax.slice_in_dim(m, 0, 1, axis=axis)],
                            axis=axis)
    shape = list(m.shape)
    shape[axis:axis + 1] = [shape[axis] // 2, 2]
    m = m.reshape(shape)
    return lax.index_in_dim(m, 0, axis=axis + 1, keepdims=False)

def _conv_body(x_ref, w_ref, b_ref, o_ref, acc_ref, *, kh, kw, ho, wo, nb, co,
               pool, pack):
    c = x_ref.shape[-1]
    acc_ref[...] = jnp.zeros_like(acc_ref)

    if pack:
        prs = (kw + 1) // 2

        def tap(q, carry):
            a, pr = q // prs, q % prs
            x0 = x_ref[pl.ds(a, ho), pl.ds(2 * pr, wo), :, :]
            x1 = x_ref[pl.ds(a, ho), pl.ds(2 * pr + 1, wo), :, :]
            xs = jnp.concatenate([x0, x1], axis=-1)
            xs = xs.reshape(ho * wo * nb, 2 * c)
            acc_ref[...] += _mm1(xs, w_ref[q, :, :])
            return carry

        lax.fori_loop(0, kh * prs, tap, 0)
    else:
        def tap(t, carry):
            a, b = t // kw, t % kw
            xs = x_ref[pl.ds(a, ho), pl.ds(b, wo), :, :]
            xs = xs.reshape(ho * wo * nb, c)
            acc_ref[...] += _mm1(xs, w_ref[t, :, :])
            return carry

        lax.fori_loop(0, kh * kw, tap, 0)
    y = jnp.maximum(acc_ref[...] + b_ref[...], 0.0)
    y = y.reshape(ho, wo, nb, co)
    if pool:
        y = _pool_axis(_pool_axis(y, 0), 1)
    o_ref[...] = y


def _conv(x, w, b, kh, kw, nb, pool, pack=False):
    hp, wp, n, c = x.shape
    t, ck, co = w.shape
    ho, wo = hp - kh + 1, wp - kw + 1
    if pack:
        wo = wp - kw
    if pool:
        oh = ow = (ho - 3) // 2 + 1
    else:
        oh, ow = ho, wo
    body = functools.partial(_conv_body, kh=kh, kw=kw, ho=ho, wo=wo, nb=nb,
                             co=co, pool=pool, pack=pack)
    return pl.pallas_call(
        body,
        grid=(n // nb,),
        in_specs=[
            pl.BlockSpec((hp, wp, nb, c), lambda i: (0, 0, i, 0)),
            pl.BlockSpec((t, ck, co), lambda i: (0, 0, 0)),
            pl.BlockSpec((1, co), lambda i: (0, 0)),
        ],
        out_specs=pl.BlockSpec((oh, ow, nb, co), lambda i: (0, 0, i, 0)),
        out_shape=jax.ShapeDtypeStruct((oh, ow, n, co), _F32),
        scratch_shapes=[pltpu.VMEM((ho * wo * nb, co), _F32)],
    )(x, w, b.reshape(1, co))


def _conv1_body(xa_ref, xb_ref, w_ref, b_ref, o_ref, acc_ref, *, hb, wo, nb,
                co):
    c = xa_ref.shape[-1]
    acc_ref[...] = jnp.zeros_like(acc_ref)

    def sl(t):
        a, b = t // 3, t % 3
        if a == 0:
            return xa_ref[:, b:b + wo, :, :]
        return jnp.concatenate(
            [xa_ref[a:hb, b:b + wo, :, :], xb_ref[0:a, b:b + wo, :, :]],
            axis=0)

    for q in range(5):
        x0 = sl(2 * q)
        x1 = sl(2 * q + 1) if 2 * q + 1 < 9 else x0
        xs = jnp.concatenate([x0, x1], axis=-1).reshape(hb * wo * nb, 2 * c)
        acc_ref[...] += _mm1(xs, w_ref[q, :, :])
    y = jnp.maximum(acc_ref[...] + b_ref[...], 0.0)
    o_ref[...] = y.reshape(hb, wo, nb, co)


def _conv1(x, w, b, nb, hb=8):
    hp, wp, n, c = x.shape
    t, ck, co = w.shape
    wo = wp - 2
    nh = hp // hb - 1
    body = functools.partial(_conv1_body, hb=hb, wo=wo, nb=nb, co=co)
    return pl.pallas_call(
        body,
        grid=(n // nb, nh),
        in_specs=[
            pl.BlockSpec((hb, wp, nb, c), lambda i, j: (j, 0, i, 0)),
            pl.BlockSpec((hb, wp, nb, c), lambda i, j: (j + 1, 0, i, 0)),
            pl.BlockSpec((t, ck, co), lambda i, j: (0, 0, 0)),
            pl.BlockSpec((1, co), lambda i, j: (0, 0)),
        ],
        out_specs=pl.BlockSpec((hb, wo, nb, co), lambda i, j: (j, 0, i, 0)),
        out_shape=jax.ShapeDtypeStruct((nh * hb, wo, n, co), _F32),
        scratch_shapes=[pltpu.VMEM((hb * wo * nb, co), _F32)],
    )(x, x, w, b.reshape(1, co))


def _pool_body(x_ref, o_ref):
    o_ref[...] = _pool_axis(_pool_axis(x_ref[...], 0), 1)


def _pool(x, nb):
    h, w, n, c = x.shape
    ph, pw = (h - 3) // 2 + 1, (w - 3) // 2 + 1
    return pl.pallas_call(
        _pool_body,
        grid=(n // nb,),
        in_specs=[pl.BlockSpec((h, w, nb, c), lambda i: (0, 0, i, 0))],
        out_specs=pl.BlockSpec((ph, pw, nb, c), lambda i: (0, 0, i, 0)),
        out_shape=jax.ShapeDtypeStruct((ph, pw, n, c), _F32),
    )(x)


def _fc(x, w, b, relu, bo=None, bk=None):
    m, kdim = x.shape
    o = w.shape[0]
    bo = bo or o
    bk = bk or kdim
    no, nk = o // bo, kdim // bk

    def body(x_ref, w_ref, b_ref, o_ref, acc_ref):
        kk = pl.program_id(1)
        part = _mm1(x_ref[...], w_ref[...], _DIMS_T)

        @pl.when(kk == 0)
        def _():
            acc_ref[...] = part

        @pl.when(kk > 0)
        def _():
            acc_ref[...] += part

        @pl.when(kk == nk - 1)
        def _():
            y = acc_ref[...] + b_ref[...]
            if relu:
                y = jnp.maximum(y, 0.0)
            o_ref[...] = y

    return pl.pallas_call(
        body,
        grid=(no, nk),
        in_specs=[
            pl.BlockSpec((m, bk), lambda i, j: (0, j)),
            pl.BlockSpec((bo, bk), lambda i, j: (i, j)),
            pl.BlockSpec((1, bo), lambda i, j: (0, i)),
        ],
        out_specs=pl.BlockSpec((m, bo), lambda i, j: (0, i)),
        out_shape=jax.ShapeDtypeStruct((m, o), _F32),
        scratch_shapes=[pltpu.VMEM((m, bo), _F32)],
    )(x, w, b.reshape(1, o))


def _tail_body(h_ref, c_ref, flw_ref, flb_ref, e_ref, d1w_ref, d1b_ref,
               d2w_ref, d2b_ref, d3w_ref, d3b_ref, d4w_ref, d4b_ref,
               d5w_ref, d5b_ref, d6w_ref, d6b_ref,
               loss_ref, xr_ref, perp_ref):
    n = h_ref.shape[0]
    hc = jnp.concatenate([h_ref[...], c_ref[...]], axis=-1)
    x = _mm1(hc, flw_ref[...], _DIMS_T) + flb_ref[...]
    e = e_ref[...]
    ne, dm = e.shape
    x2 = jnp.sum(x * x, axis=1, keepdims=True)
    e2 = lax.dot_general(jnp.ones((1, dm), _F32), e * e, _DIMS_T,
                         precision=lax.Precision.HIGHEST,
                         preferred_element_type=_F32)
    xe = _mm1(x, e, _DIMS_T)
    d = x2 + e2 - 2.0 * xe
    iota = lax.broadcasted_iota(jnp.int32, (n, ne), 1)
    dmin = jnp.min(d, axis=1, keepdims=True)
    idx = jnp.min(jnp.where(d == dmin, iota, ne), axis=1, keepdims=True)
    enc = (iota == idx).astype(_F32)
    q = _mm1(enc, e)
    diff = q - x
    ss = jnp.sum(jnp.sum(diff * diff, axis=1, keepdims=True), axis=0,
                 keepdims=True)
    loss_ref[...] = 0.25 * ss / (n * dm)
    avg = jnp.sum(enc, axis=0, keepdims=True) / n
    ent = jnp.sum(avg * jnp.log(avg + 1e-10), axis=1, keepdims=True)
    perp_ref[...] = jnp.exp(-ent)
    v = jnp.maximum(_mm1(q, d1w_ref[...], _DIMS_T) + d1b_ref[...], 0.0)
    v = jnp.maximum(_mm1(v, d2w_ref[...], _DIMS_T) + d2b_ref[...], 0.0)
    c2 = jnp.maximum(_mm1(c_ref[...], d3w_ref[...], _DIMS_T) + d3b_ref[...],
                     0.0)
    v = jnp.concatenate([v, c2], axis=-1)
    v = jnp.maximum(_mm1(v, d4w_ref[...], _DIMS_T) + d4b_ref[...], 0.0)
    v = jnp.maximum(_mm1(v, d5w_ref[...], _DIMS_T) + d5b_ref[...], 0.0)
    xr_ref[...] = _mm1(v, d6w_ref[...], _DIMS_T) + d6b_ref[...]


def _tail(h, c, p):
    n = h.shape[0]
    args = [h, c, p["e_flw"], p["e_flb"].reshape(1, -1), p["emb"]]
    for k in ("d_fc1", "d_fc2", "d_fc3", "d_fc4", "d_fc5", "d_fc6"):
        args += [p[k + "w"], p[k + "b"].reshape(1, -1)]
    loss, xr, perp = pl.pallas_call(
        _tail_body,
        out_shape=(jax.ShapeDtypeStruct((1, 1), _F32),
                   jax.ShapeDtypeStruct((n, 72), _F32),
                   jax.ShapeDtypeStruct((1, 1), _F32)),
    )(*args)
    return loss.reshape(()), xr, perp.reshape(())


def _vq(latent, emb):
    n, dm = latent.shape
    ne = emb.shape[0]

    def body(x_ref, e_ref, loss_ref, q_ref, perp_ref):
        x = x_ref[...]
        e = e_ref[...]
        x2 = jnp.sum(x * x, axis=1, keepdims=True)
        e2 = lax.dot_general(jnp.ones((1, dm), _F32), e * e, _DIMS_T,
                             precision=lax.Precision.HIGHEST,
                             preferred_element_type=_F32)
        xe = _mm1(x, e, _DIMS_T)
        d = x2 + e2 - 2.0 * xe
        iota = lax.broadcasted_iota(jnp.int32, (n, ne), 1)
        dmin = jnp.min(d, axis=1, keepdims=True)
        idx = jnp.min(jnp.where(d == dmin, iota, ne), axis=1, keepdims=True)
        enc = (iota == idx).astype(_F32)
        q = _mm1(enc, e)
        diff = q - x
        ss = jnp.sum(jnp.sum(diff * diff, axis=1, keepdims=True), axis=0,
                     keepdims=True)
        loss_ref[...] = 0.25 * ss / (n * dm)
        q_ref[...] = q
        avg = jnp.sum(enc, axis=0, keepdims=True) / n
        ent = jnp.sum(avg * jnp.log(avg + 1e-10), axis=1, keepdims=True)
        perp_ref[...] = jnp.exp(-ent)

    loss, q, perp = pl.pallas_call(
        body,
        out_shape=(jax.ShapeDtypeStruct((1, 1), _F32),
                   jax.ShapeDtypeStruct((n, dm), _F32),
                   jax.ShapeDtypeStruct((1, 1), _F32)),
    )(latent, emb)
    return loss.reshape(()), q, perp.reshape(())


def kernel(x, pose, img, img_crop, img_zoom, params):
    p = params
    imgs = jnp.concatenate([img, img_crop, img_zoom], axis=0)
    imgs = jnp.pad(imgs, ((0, 0), (0, 1), (2, 30), (2, 2)))
    xd = imgs.reshape(48, 4, 64, 4, 57, 4).transpose(2, 4, 0, 3, 5, 1)
    xd = xd.reshape(64, 57, 48, 64)
    w1 = jnp.pad(p["c1w"], ((0, 0), (0, 1), (0, 1), (0, 1)))
    w1 = w1.reshape(64, 4, 3, 4, 3, 4).transpose(2, 4, 3, 5, 1, 0)
    w1 = jnp.pad(w1.reshape(9, 64, 64), ((0, 1), (0, 0), (0, 0)))
    w1 = w1.reshape(5, 128, 64)
    y = _conv1(xd, w1, p["c1b"], nb=8)
    y = _pool(y, nb=8)
    y = jnp.pad(y, ((2, 2), (2, 3), (0, 0), (0, 0)))
    w2 = p["c2w"].transpose(2, 3, 1, 0)
    w2 = jnp.pad(w2, ((0, 0), (0, 1), (0, 0), (0, 0)))
    w2 = w2.reshape(15, 128, 192)
    y = _conv(y, w2, p["c2b"], 5, 5, nb=8, pool=True, pack=True)
    y = jnp.pad(y, ((1, 1), (1, 1), (0, 0), (0, 0)))
    w3 = p["c3w"].transpose(2, 3, 1, 0).reshape(9, 192, 384)
    y = _conv(y, w3, p["c3b"], 3, 3, nb=16, pool=False)
    y = jnp.pad(y, ((1, 1), (1, 1), (0, 0), (0, 0)))
    w4 = p["c4w"].transpose(2, 3, 1, 0).reshape(9, 384, 256)
    y = _conv(y, w4, p["c4b"], 3, 3, nb=16, pool=False)
    y = jnp.pad(y, ((1, 1), (1, 1), (0, 0), (0, 0)))
    w5 = p["c5w"].transpose(2, 3, 1, 0).reshape(9, 256, 256)
    y = _conv(y, w5, p["c5b"], 3, 3, nb=16, pool=True)
    feat = y.transpose(2, 3, 0, 1).reshape(48, 9216)
    f = _fc(feat, p["fc6w"], p["fc6b"], True, bo=512, bk=2304)
    f = _fc(f, p["fc7w"], p["fc7b"], True, bo=512, bk=2048)
    f1, f2, f3 = f[0:16], f[16:32], f[32:48]
    pf = _fc(pose, p["ce_fc1w"], p["ce_fc1b"], True)
    hcat = jnp.concatenate([pf, f1, f2, f3], axis=1)
    c = _fc(hcat, p["ce_fc2w"], p["ce_fc2b"], True, bo=512, bk=3328)
    h = _fc(x, p["e_fc1w"], p["e_fc1b"], True)
    h = _fc(h, p["e_fc2w"], p["e_fc2b"], True)
    loss, xr, perp = _tail(h, c, p)
    return loss, xr, perp
